# baseline (speedup 1.0000x reference)
import functools

import jax
import jax.numpy as jnp
from jax import lax
from jax.experimental import pallas as pl
from jax.experimental.pallas import tpu as pltpu
from jax.experimental.pallas import tpu_sc as plsc

_NUM_PIECES = 32
_C = 128
_HW = 64
_ROW = _HW + 1
_PAD_W = _C * _ROW
_BROWS = _C * _HW // 128
_OROWS = _NUM_PIECES
_NW = 32
_K = 4


def _ctz32(x):
    low = x & (0 - x)
    is_top = low == jnp.int32(-2147483648)
    f = low.astype(jnp.float32)
    e = (plsc.bitcast(f, jnp.int32) >> 23) & 0xFF
    return jnp.where(is_top, jnp.int32(31), e - 127)


def _lane_bcast(v, lane):
    idx = jnp.full((16, 1), lane, jnp.int32)
    return lax.gather(
        v, idx,
        lax.GatherDimensionNumbers(
            offset_dims=(), collapsed_slice_dims=(0,), start_index_map=(0,)),
        (1,), mode=lax.GatherScatterMode.PROMISE_IN_BOUNDS)


def _sc_body(board_hbm, ids_hbm, out_hbm, stage, idsv, outv, bufp,
             occ_lo, occ_hi):
    B = board_hbm.shape[0] // _BROWS
    bpw = B // _NW
    nch = bpw // _K
    cid = lax.axis_index("c")
    sid = lax.axis_index("s")
    wid = sid * 2 + cid
    base = wid * bpw
    iota = lax.iota(jnp.int32, 16)
    zf16 = jnp.zeros((16,), jnp.float32)
    zi16 = jnp.zeros((16,), jnp.int32)

    for k in range(_K):
        for c0 in range(0, _C, 16):
            plsc.store_scatter(
                bufp, [k * _PAD_W + (c0 + iota) * _ROW + _HW], zf16)

    def chunk_body(ch, carry):
        b0 = base + ch * _K
        pltpu.sync_copy(board_hbm.at[pl.ds(b0 * _BROWS, _K * _BROWS)], stage)
        pltpu.sync_copy(ids_hbm.at[pl.ds(b0 * _HW, _K * _HW)], idsv)

        def board_body(k, carry2):
            kpad = k * _PAD_W
            for r in range(_BROWS):
                for l0 in range(0, 128, 16):
                    c = 2 * r + l0 // 64
                    hw0 = l0 % 64
                    g = stage[k * _BROWS + r, pl.ds(l0, 16)]
                    bufp[pl.ds(kpad + c * _ROW + hw0, 16)] = g

            for seg in range(3):
                occ_lo[pl.ds(16 * seg, 16)] = zi16
                occ_hi[pl.ds(16 * seg, 16)] = zi16
            for j in range(4):
                idv = idsv[pl.ds(k * _HW + 16 * j, 16)]
                bits = jnp.int32(1) << (iota + (16 if j % 2 else 0))
                plsc.addupdate_scatter(occ_lo if j < 2 else occ_hi,
                                       [idv], bits)
            lo_a = plsc.load_gather(occ_lo, [iota + 1])
            hi_a = plsc.load_gather(occ_hi, [iota + 1])
            lo_b = plsc.load_gather(occ_lo, [iota + 17])
            hi_b = plsc.load_gather(occ_hi, [iota + 17])

            def first_of(lo, hi):
                return jnp.where(
                    lo != 0, _ctz32(lo),
                    jnp.where(hi != 0, _ctz32(hi) + 32, jnp.int32(_HW)))

            first_a = first_of(lo_a, hi_a)
            first_b = first_of(lo_b, hi_b)

            for t in range(_NUM_PIECES):
                fv = _lane_bcast(first_a if t < 16 else first_b, t % 16)
                fvk = fv + kpad
                for c0 in range(0, _C, 16):
                    g = plsc.load_gather(bufp, [fvk + (c0 + iota) * _ROW])
                    outv[k * _OROWS + t, pl.ds(c0, 16)] = g
            return carry2

        lax.fori_loop(0, _K, board_body, 0)
        pltpu.sync_copy(outv, out_hbm.at[pl.ds(b0 * _OROWS, _K * _OROWS)])
        return carry

    lax.fori_loop(0, nch, chunk_body, 0)


def kernel(full_board_vector, piece_ids):
    B, C, H, W = full_board_vector.shape
    board2d = full_board_vector.reshape(B * _BROWS, 128)
    ids_flat = piece_ids.reshape(B * H * W)

    mesh = plsc.VectorSubcoreMesh(core_axis_name="c", subcore_axis_name="s")
    run = functools.partial(
        pl.kernel,
        out_type=jax.ShapeDtypeStruct((B * _OROWS, 128), jnp.float32),
        mesh=mesh,
        compiler_params=pltpu.CompilerParams(needs_layout_passes=False),
        scratch_types=[
            pltpu.VMEM((_K * _BROWS, 128), jnp.float32),
            pltpu.VMEM((_K * _HW,), jnp.int32),
            pltpu.VMEM((_K * _OROWS, 128), jnp.float32),
            pltpu.VMEM((_K * _PAD_W,), jnp.float32),
            pltpu.VMEM((48,), jnp.int32),
            pltpu.VMEM((48,), jnp.int32),
        ],
    )(_sc_body)
    return run(board2d, ids_flat).reshape(B, _NUM_PIECES, C)


# speedup vs baseline: 14.3262x; 14.3262x over previous
import functools

import jax
import jax.numpy as jnp
from jax import lax
from jax.experimental import pallas as pl
from jax.experimental.pallas import tpu as pltpu
from jax.experimental.pallas import tpu_sc as plsc

_NUM_PIECES = 32
_C = 128
_HW = 64
_NW = 32
_K = 4


def _ctz32(x):
    low = x & (0 - x)
    is_top = low == jnp.int32(-2147483648)
    f = low.astype(jnp.float32)
    e = (plsc.bitcast(f, jnp.int32) >> 23) & 0xFF
    return jnp.where(is_top, jnp.int32(31), e - 127)


def _sc_body(table_hbm, ids_hbm, out_hbm, idsv, rows, occ_lo, occ_hi,
             aflags, sem):
    B = table_hbm.shape[0] // _HW
    bpw = B // _NW
    nch = bpw // _K
    cid = lax.axis_index("c")
    sid = lax.axis_index("s")
    wid = sid * 2 + cid
    base = wid * bpw
    iota = lax.iota(jnp.int32, 16)
    zf16 = jnp.zeros((16,), jnp.float32)
    zi16 = jnp.zeros((16,), jnp.int32)

    def chunk_body(ch, carry):
        b0 = base + ch * _K
        pltpu.sync_copy(ids_hbm.at[pl.ds(b0 * _HW, _K * _HW)], idsv)

        def board_body(k, carry2):
            for seg in range(3):
                occ_lo[pl.ds(16 * seg, 16)] = zi16
                occ_hi[pl.ds(16 * seg, 16)] = zi16
            for j in range(4):
                idv = idsv[pl.ds(k * _HW + 16 * j, 16)]
                bits = jnp.int32(1) << (iota + (16 if j % 2 else 0))
                plsc.addupdate_scatter(occ_lo if j < 2 else occ_hi,
                                       [idv], bits)
            lo_a = plsc.load_gather(occ_lo, [iota + 1])
            hi_a = plsc.load_gather(occ_hi, [iota + 1])
            lo_b = plsc.load_gather(occ_lo, [iota + 17])
            hi_b = plsc.load_gather(occ_hi, [iota + 17])

            def first_of(lo, hi):
                return jnp.where(
                    lo != 0, _ctz32(lo),
                    jnp.where(hi != 0, _ctz32(hi) + 32, jnp.int32(0)))

            first_a = first_of(lo_a, hi_a)
            first_b = first_of(lo_b, hi_b)
            row0 = (b0 + k) * _HW
            pltpu.async_copy(table_hbm.at[row0 + first_a],
                             rows.at[pl.ds(k * _NUM_PIECES, 16)], sem)
            pltpu.async_copy(table_hbm.at[row0 + first_b],
                             rows.at[pl.ds(k * _NUM_PIECES + 16, 16)], sem)
            aflags[pl.ds(k * _NUM_PIECES, 16)] = \
                ((lo_a | hi_a) == 0).astype(jnp.int32)
            aflags[pl.ds(k * _NUM_PIECES + 16, 16)] = \
                ((lo_b | hi_b) == 0).astype(jnp.int32)
            return carry2

        lax.fori_loop(0, _K, board_body, 0)
        for k in range(_K):
            pltpu.make_async_copy(
                table_hbm.at[pl.ds(0, 16)],
                rows.at[pl.ds(k * _NUM_PIECES, 16)], sem).wait()
            pltpu.make_async_copy(
                table_hbm.at[pl.ds(0, 16)],
                rows.at[pl.ds(k * _NUM_PIECES + 16, 16)], sem).wait()

        def absent_fix(k, carry3):
            ab_a = aflags[pl.ds(k * _NUM_PIECES, 16)]
            ab_b = aflags[pl.ds(k * _NUM_PIECES + 16, 16)]
            for t in range(_NUM_PIECES):
                av = ab_a if t < 16 else ab_b
                flag = jnp.max(jnp.where(iota == t % 16, av, 0))

                @pl.when(flag != 0)
                def _zero_row():
                    r = k * _NUM_PIECES + t
                    for c0 in range(0, _C, 16):
                        rows[r, pl.ds(c0, 16)] = zf16
            return carry3

        lax.fori_loop(0, _K, absent_fix, 0)
        pltpu.sync_copy(rows,
                        out_hbm.at[pl.ds(b0 * _NUM_PIECES,
                                         _K * _NUM_PIECES)])
        return carry

    lax.fori_loop(0, nch, chunk_body, 0)


def kernel(full_board_vector, piece_ids):
    B, C, H, W = full_board_vector.shape
    table = full_board_vector.transpose(0, 2, 3, 1).reshape(B * H * W, C)
    ids_flat = piece_ids.reshape(B * H * W)

    mesh = plsc.VectorSubcoreMesh(core_axis_name="c", subcore_axis_name="s")
    run = functools.partial(
        pl.kernel,
        out_type=jax.ShapeDtypeStruct((B * _NUM_PIECES, C), jnp.float32),
        mesh=mesh,
        compiler_params=pltpu.CompilerParams(needs_layout_passes=False),
        scratch_types=[
            pltpu.VMEM((_K * _HW,), jnp.int32),
            pltpu.VMEM((_K * _NUM_PIECES, _C), jnp.float32),
            pltpu.VMEM((48,), jnp.int32),
            pltpu.VMEM((48,), jnp.int32),
            pltpu.VMEM((_K * _NUM_PIECES,), jnp.int32),
            pltpu.SemaphoreType.DMA,
        ],
    )(_sc_body)
    return run(table, ids_flat).reshape(B, _NUM_PIECES, C)


# speedup vs baseline: 16.6391x; 1.1614x over previous
import functools

import jax
import jax.numpy as jnp
from jax import lax
from jax.experimental import pallas as pl
from jax.experimental.pallas import tpu as pltpu
from jax.experimental.pallas import tpu_sc as plsc

_NUM_PIECES = 32
_C = 128
_HW = 64
_NW = 32
_K = 8


def _ctz32(x):
    low = x & (0 - x)
    is_top = low == jnp.int32(-2147483648)
    f = low.astype(jnp.float32)
    e = (plsc.bitcast(f, jnp.int32) >> 23) & 0xFF
    return jnp.where(is_top, jnp.int32(31), e - 127)


def _sc_body(table_hbm, ids_hbm, out_hbm, idsv, rows, occ_lo, occ_hi,
             aflags, sem):
    B = table_hbm.shape[0] // _HW
    bpw = B // _NW
    nch = bpw // _K
    cid = lax.axis_index("c")
    sid = lax.axis_index("s")
    wid = sid * 2 + cid
    base = wid * bpw
    iota = lax.iota(jnp.int32, 16)
    zf16 = jnp.zeros((16,), jnp.float32)
    zi16 = jnp.zeros((16,), jnp.int32)

    def chunk_body(ch, carry):
        b0 = base + ch * _K
        pltpu.sync_copy(ids_hbm.at[pl.ds(b0 * _HW, _K * _HW)], idsv)

        def board_body(k, carry2):
            for seg in range(3):
                occ_lo[pl.ds(16 * seg, 16)] = zi16
                occ_hi[pl.ds(16 * seg, 16)] = zi16
            for j in range(4):
                idv = idsv[pl.ds(k * _HW + 16 * j, 16)]
                bits = jnp.int32(1) << (iota + (16 if j % 2 else 0))
                plsc.addupdate_scatter(occ_lo if j < 2 else occ_hi,
                                       [idv], bits)
            lo_a = plsc.load_gather(occ_lo, [iota + 1])
            hi_a = plsc.load_gather(occ_hi, [iota + 1])
            lo_b = plsc.load_gather(occ_lo, [iota + 17])
            hi_b = plsc.load_gather(occ_hi, [iota + 17])

            def first_of(lo, hi):
                return jnp.where(
                    lo != 0, _ctz32(lo),
                    jnp.where(hi != 0, _ctz32(hi) + 32, jnp.int32(0)))

            first_a = first_of(lo_a, hi_a)
            first_b = first_of(lo_b, hi_b)
            row0 = (b0 + k) * _HW
            pltpu.async_copy(table_hbm.at[row0 + first_a],
                             rows.at[pl.ds(k * _NUM_PIECES, 16)], sem)
            pltpu.async_copy(table_hbm.at[row0 + first_b],
                             rows.at[pl.ds(k * _NUM_PIECES + 16, 16)], sem)
            bits_a = jnp.where(lo_a | hi_a, 0, jnp.int32(1) << iota)
            bits_b = jnp.where(lo_b | hi_b, 0, jnp.int32(1) << (iota + 16))
            aflags[pl.ds(k * 16, 16)] = bits_a | bits_b
            return carry2

        lax.fori_loop(0, _K, board_body, 0)
        for k in range(_K):
            pltpu.make_async_copy(
                table_hbm.at[pl.ds(0, 16)],
                rows.at[pl.ds(k * _NUM_PIECES, 16)], sem).wait()
            pltpu.make_async_copy(
                table_hbm.at[pl.ds(0, 16)],
                rows.at[pl.ds(k * _NUM_PIECES + 16, 16)], sem).wait()

        def absent_fix(k, carry3):
            av = aflags[pl.ds(k * 16, 16)]
            bits = jnp.sum(av)
            for t in range(_NUM_PIECES):
                flag = (bits >> t) & 1

                @pl.when(flag != 0)
                def _zero_row():
                    r = k * _NUM_PIECES + t
                    for c0 in range(0, _C, 16):
                        rows[r, pl.ds(c0, 16)] = zf16
            return carry3

        lax.fori_loop(0, _K, absent_fix, 0)
        pltpu.sync_copy(rows,
                        out_hbm.at[pl.ds(b0 * _NUM_PIECES,
                                         _K * _NUM_PIECES)])
        return carry

    lax.fori_loop(0, nch, chunk_body, 0)


def kernel(full_board_vector, piece_ids):
    B, C, H, W = full_board_vector.shape
    table = full_board_vector.transpose(0, 2, 3, 1).reshape(B * H * W, C)
    ids_flat = piece_ids.reshape(B * H * W)

    mesh = plsc.VectorSubcoreMesh(core_axis_name="c", subcore_axis_name="s")
    run = functools.partial(
        pl.kernel,
        out_type=jax.ShapeDtypeStruct((B * _NUM_PIECES, C), jnp.float32),
        mesh=mesh,
        compiler_params=pltpu.CompilerParams(needs_layout_passes=False),
        scratch_types=[
            pltpu.VMEM((_K * _HW,), jnp.int32),
            pltpu.VMEM((_K * _NUM_PIECES, _C), jnp.float32),
            pltpu.VMEM((48,), jnp.int32),
            pltpu.VMEM((48,), jnp.int32),
            pltpu.VMEM((_K * 16,), jnp.int32),
            pltpu.SemaphoreType.DMA,
        ],
    )(_sc_body)
    return run(table, ids_flat).reshape(B, _NUM_PIECES, C)


# speedup vs baseline: 18.1736x; 1.0922x over previous
import functools

import jax
import jax.numpy as jnp
from jax import lax
from jax.experimental import pallas as pl
from jax.experimental.pallas import tpu as pltpu
from jax.experimental.pallas import tpu_sc as plsc

_NUM_PIECES = 32
_C = 128
_HW = 64
_NW = 32
_K = 8


def _ctz32(x):
    low = x & (0 - x)
    is_top = low == jnp.int32(-2147483648)
    f = low.astype(jnp.float32)
    e = (plsc.bitcast(f, jnp.int32) >> 23) & 0xFF
    return jnp.where(is_top, jnp.int32(31), e - 127)


def _sc_body(table_hbm, ids_hbm, out_hbm, idsv0, idsv1, rows0, rows1,
             occ_lo, occ_hi, aflags, sem, osem0, osem1):
    B = table_hbm.shape[0] // _HW
    bpw = B // _NW
    nch = bpw // _K
    cid = lax.axis_index("c")
    sid = lax.axis_index("s")
    wid = sid * 2 + cid
    base = wid * bpw
    iota = lax.iota(jnp.int32, 16)
    zf16 = jnp.zeros((16,), jnp.float32)
    zi16 = jnp.zeros((16,), jnp.int32)

    def half_body(i2, h, idsv, rows, osem):
        ch = i2 * 2 + h
        b0 = base + ch * _K

        @pl.when(i2 > 0)
        def _wait_prev_out():
            pltpu.make_async_copy(
                table_hbm.at[pl.ds(0, _K * _NUM_PIECES)], rows, osem).wait()

        pltpu.sync_copy(ids_hbm.at[pl.ds(b0 * _HW, _K * _HW)], idsv)

        def board_body(k, carry2):
            for seg in range(3):
                occ_lo[pl.ds(16 * seg, 16)] = zi16
                occ_hi[pl.ds(16 * seg, 16)] = zi16
            for j in range(4):
                idv = idsv[pl.ds(k * _HW + 16 * j, 16)]
                bits = jnp.int32(1) << (iota + (16 if j % 2 else 0))
                plsc.addupdate_scatter(occ_lo if j < 2 else occ_hi,
                                       [idv], bits)
            lo_a = plsc.load_gather(occ_lo, [iota + 1])
            hi_a = plsc.load_gather(occ_hi, [iota + 1])
            lo_b = plsc.load_gather(occ_lo, [iota + 17])
            hi_b = plsc.load_gather(occ_hi, [iota + 17])

            def first_of(lo, hi):
                return jnp.where(
                    lo != 0, _ctz32(lo),
                    jnp.where(hi != 0, _ctz32(hi) + 32, jnp.int32(0)))

            first_a = first_of(lo_a, hi_a)
            first_b = first_of(lo_b, hi_b)
            row0 = (b0 + k) * _HW
            pltpu.async_copy(table_hbm.at[row0 + first_a],
                             rows.at[pl.ds(k * _NUM_PIECES, 16)], sem)
            pltpu.async_copy(table_hbm.at[row0 + first_b],
                             rows.at[pl.ds(k * _NUM_PIECES + 16, 16)], sem)
            bits_a = jnp.where(lo_a | hi_a, 0, jnp.int32(1) << iota)
            bits_b = jnp.where(lo_b | hi_b, 0, jnp.int32(1) << (iota + 16))
            aflags[pl.ds(k * 16, 16)] = bits_a | bits_b
            return carry2

        lax.fori_loop(0, _K, board_body, 0)
        for k in range(_K):
            pltpu.make_async_copy(
                table_hbm.at[pl.ds(0, 16)],
                rows.at[pl.ds(k * _NUM_PIECES, 16)], sem).wait()
            pltpu.make_async_copy(
                table_hbm.at[pl.ds(0, 16)],
                rows.at[pl.ds(k * _NUM_PIECES + 16, 16)], sem).wait()

        def absent_fix(k, carry3):
            av = aflags[pl.ds(k * 16, 16)]
            bits = jnp.sum(av)
            for t in range(_NUM_PIECES):
                flag = (bits >> t) & 1

                @pl.when(flag != 0)
                def _zero_row():
                    r = k * _NUM_PIECES + t
                    for c0 in range(0, _C, 16):
                        rows[r, pl.ds(c0, 16)] = zf16
            return carry3

        lax.fori_loop(0, _K, absent_fix, 0)
        pltpu.async_copy(rows,
                         out_hbm.at[pl.ds(b0 * _NUM_PIECES,
                                          _K * _NUM_PIECES)], osem)

    def pair_body(i2, carry):
        half_body(i2, 0, idsv0, rows0, osem0)
        half_body(i2, 1, idsv1, rows1, osem1)
        return carry

    lax.fori_loop(0, nch // 2, pair_body, 0)
    pltpu.make_async_copy(
        table_hbm.at[pl.ds(0, _K * _NUM_PIECES)], rows0, osem0).wait()
    pltpu.make_async_copy(
        table_hbm.at[pl.ds(0, _K * _NUM_PIECES)], rows1, osem1).wait()


def kernel(full_board_vector, piece_ids):
    B, C, H, W = full_board_vector.shape
    table = full_board_vector.transpose(0, 2, 3, 1).reshape(B * H * W, C)
    ids_flat = piece_ids.reshape(B * H * W)

    mesh = plsc.VectorSubcoreMesh(core_axis_name="c", subcore_axis_name="s")
    run = functools.partial(
        pl.kernel,
        out_type=jax.ShapeDtypeStruct((B * _NUM_PIECES, C), jnp.float32),
        mesh=mesh,
        compiler_params=pltpu.CompilerParams(needs_layout_passes=False),
        scratch_types=[
            pltpu.VMEM((_K * _HW,), jnp.int32),
            pltpu.VMEM((_K * _HW,), jnp.int32),
            pltpu.VMEM((_K * _NUM_PIECES, _C), jnp.float32),
            pltpu.VMEM((_K * _NUM_PIECES, _C), jnp.float32),
            pltpu.VMEM((48,), jnp.int32),
            pltpu.VMEM((48,), jnp.int32),
            pltpu.VMEM((_K * 16,), jnp.int32),
            pltpu.SemaphoreType.DMA,
            pltpu.SemaphoreType.DMA,
            pltpu.SemaphoreType.DMA,
        ],
    )(_sc_body)
    return run(table, ids_flat).reshape(B, _NUM_PIECES, C)


# speedup vs baseline: 18.3689x; 1.0107x over previous
import functools

import jax
import jax.numpy as jnp
from jax import lax
from jax.experimental import pallas as pl
from jax.experimental.pallas import tpu as pltpu
from jax.experimental.pallas import tpu_sc as plsc

_NUM_PIECES = 32
_C = 128
_HW = 64
_NW = 32
_K = 8


def _ctz32(x):
    low = x & (0 - x)
    is_top = low == jnp.int32(-2147483648)
    f = low.astype(jnp.float32)
    e = (plsc.bitcast(f, jnp.int32) >> 23) & 0xFF
    return jnp.where(is_top, jnp.int32(31), e - 127)


def _sc_body(table_hbm, ids_hbm, out_hbm, idsv, rows0, rows1,
             occ_lo, occ_hi, aflags, sem, osem0, osem1):
    B = table_hbm.shape[0] // _HW
    bpw = B // _NW
    nch = bpw // _K
    cid = lax.axis_index("c")
    sid = lax.axis_index("s")
    wid = sid * 2 + cid
    base = wid * bpw
    iota = lax.iota(jnp.int32, 16)
    zf16 = jnp.zeros((16,), jnp.float32)
    zi16 = jnp.zeros((16,), jnp.int32)

    pltpu.sync_copy(ids_hbm.at[pl.ds(base * _HW, bpw * _HW)], idsv)

    def half_body(i2, h, rows, osem):
        ch = i2 * 2 + h
        b0 = base + ch * _K
        ids_off = ch * _K * _HW

        @pl.when(i2 > 0)
        def _wait_prev_out():
            pltpu.make_async_copy(
                table_hbm.at[pl.ds(0, _K * _NUM_PIECES)], rows, osem).wait()

        def board_body(k, carry2):
            for seg in range(3):
                occ_lo[pl.ds(16 * seg, 16)] = zi16
                occ_hi[pl.ds(16 * seg, 16)] = zi16
            for j in range(4):
                idv = idsv[pl.ds(ids_off + k * _HW + 16 * j, 16)]
                bits = jnp.int32(1) << (iota + (16 if j % 2 else 0))
                plsc.addupdate_scatter(occ_lo if j < 2 else occ_hi,
                                       [idv], bits)
            lo_a = plsc.load_gather(occ_lo, [iota + 1])
            hi_a = plsc.load_gather(occ_hi, [iota + 1])
            lo_b = plsc.load_gather(occ_lo, [iota + 17])
            hi_b = plsc.load_gather(occ_hi, [iota + 17])

            def first_of(lo, hi):
                return jnp.where(
                    lo != 0, _ctz32(lo),
                    jnp.where(hi != 0, _ctz32(hi) + 32, jnp.int32(0)))

            first_a = first_of(lo_a, hi_a)
            first_b = first_of(lo_b, hi_b)
            row0 = (b0 + k) * _HW
            pltpu.async_copy(table_hbm.at[row0 + first_a],
                             rows.at[pl.ds(k * _NUM_PIECES, 16)], sem)
            pltpu.async_copy(table_hbm.at[row0 + first_b],
                             rows.at[pl.ds(k * _NUM_PIECES + 16, 16)], sem)
            bits_a = jnp.where(lo_a | hi_a, 0, jnp.int32(1) << iota)
            bits_b = jnp.where(lo_b | hi_b, 0, jnp.int32(1) << (iota + 16))
            aflags[pl.ds(k * 16, 16)] = bits_a | bits_b
            return carry2

        lax.fori_loop(0, _K, board_body, 0)
        for k in range(_K):
            pltpu.make_async_copy(
                table_hbm.at[pl.ds(0, 16)],
                rows.at[pl.ds(k * _NUM_PIECES, 16)], sem).wait()
            pltpu.make_async_copy(
                table_hbm.at[pl.ds(0, 16)],
                rows.at[pl.ds(k * _NUM_PIECES + 16, 16)], sem).wait()

        def absent_fix(k, carry3):
            av = aflags[pl.ds(k * 16, 16)]
            bits = jnp.sum(av)
            for t in range(_NUM_PIECES):
                flag = (bits >> t) & 1

                @pl.when(flag != 0)
                def _zero_row():
                    r = k * _NUM_PIECES + t
                    for c0 in range(0, _C, 16):
                        rows[r, pl.ds(c0, 16)] = zf16
            return carry3

        lax.fori_loop(0, _K, absent_fix, 0)
        pltpu.async_copy(rows,
                         out_hbm.at[pl.ds(b0 * _NUM_PIECES,
                                          _K * _NUM_PIECES)], osem)

    def pair_body(i2, carry):
        half_body(i2, 0, rows0, osem0)
        half_body(i2, 1, rows1, osem1)
        return carry

    lax.fori_loop(0, nch // 2, pair_body, 0)
    pltpu.make_async_copy(
        table_hbm.at[pl.ds(0, _K * _NUM_PIECES)], rows0, osem0).wait()
    pltpu.make_async_copy(
        table_hbm.at[pl.ds(0, _K * _NUM_PIECES)], rows1, osem1).wait()


def kernel(full_board_vector, piece_ids):
    B, C, H, W = full_board_vector.shape
    table = full_board_vector.transpose(0, 2, 3, 1).reshape(B * H * W, C)
    ids_flat = piece_ids.reshape(B * H * W)

    mesh = plsc.VectorSubcoreMesh(core_axis_name="c", subcore_axis_name="s")
    run = functools.partial(
        pl.kernel,
        out_type=jax.ShapeDtypeStruct((B * _NUM_PIECES, C), jnp.float32),
        mesh=mesh,
        compiler_params=pltpu.CompilerParams(needs_layout_passes=False),
        scratch_types=[
            pltpu.VMEM((B * H * W // _NW,), jnp.int32),
            pltpu.VMEM((_K * _NUM_PIECES, _C), jnp.float32),
            pltpu.VMEM((_K * _NUM_PIECES, _C), jnp.float32),
            pltpu.VMEM((48,), jnp.int32),
            pltpu.VMEM((48,), jnp.int32),
            pltpu.VMEM((_K * 16,), jnp.int32),
            pltpu.SemaphoreType.DMA,
            pltpu.SemaphoreType.DMA,
            pltpu.SemaphoreType.DMA,
        ],
    )(_sc_body)
    return run(table, ids_flat).reshape(B, _NUM_PIECES, C)


# speedup vs baseline: 22.7986x; 1.2412x over previous
import functools

import jax
import jax.numpy as jnp
from jax import lax
from jax.experimental import pallas as pl
from jax.experimental.pallas import tpu as pltpu
from jax.experimental.pallas import tpu_sc as plsc

_NUM_PIECES = 32
_C = 128
_HW = 64
_NW = 32
_K = 8


def _ctz32(x):
    low = x & (0 - x)
    is_top = low == jnp.int32(-2147483648)
    f = low.astype(jnp.float32)
    e = (plsc.bitcast(f, jnp.int32) >> 23) & 0xFF
    return jnp.where(is_top, jnp.int32(31), e - 127)


def _sc_body(table_hbm, ids_hbm, out_hbm, idsv, rows0, rows1,
             occ_lo, occ_hi, afl0, afl1, gsem0, gsem1, osem0, osem1):
    B = table_hbm.shape[0] // _HW
    bpw = B // _NW
    nch = bpw // _K
    cid = lax.axis_index("c")
    sid = lax.axis_index("s")
    wid = sid * 2 + cid
    base = wid * bpw
    iota = lax.iota(jnp.int32, 16)
    zf16 = jnp.zeros((16,), jnp.float32)
    zi16 = jnp.zeros((16,), jnp.int32)

    pltpu.sync_copy(ids_hbm.at[pl.ds(base * _HW, bpw * _HW)], idsv)

    def issue_stage(ch, rows, gsem, aflags):
        b0 = base + ch * _K
        ids_off = ch * _K * _HW

        def board_body(k, carry2):
            for seg in range(3):
                occ_lo[pl.ds(16 * seg, 16)] = zi16
                occ_hi[pl.ds(16 * seg, 16)] = zi16
            for j in range(4):
                idv = idsv[pl.ds(ids_off + k * _HW + 16 * j, 16)]
                bits = jnp.int32(1) << (iota + (16 if j % 2 else 0))
                plsc.addupdate_scatter(occ_lo if j < 2 else occ_hi,
                                       [idv], bits)
            lo_a = plsc.load_gather(occ_lo, [iota + 1])
            hi_a = plsc.load_gather(occ_hi, [iota + 1])
            lo_b = plsc.load_gather(occ_lo, [iota + 17])
            hi_b = plsc.load_gather(occ_hi, [iota + 17])

            def first_of(lo, hi):
                return jnp.where(
                    lo != 0, _ctz32(lo),
                    jnp.where(hi != 0, _ctz32(hi) + 32, jnp.int32(0)))

            first_a = first_of(lo_a, hi_a)
            first_b = first_of(lo_b, hi_b)
            row0 = (b0 + k) * _HW
            pltpu.async_copy(table_hbm.at[row0 + first_a],
                             rows.at[pl.ds(k * _NUM_PIECES, 16)], gsem)
            pltpu.async_copy(table_hbm.at[row0 + first_b],
                             rows.at[pl.ds(k * _NUM_PIECES + 16, 16)], gsem)
            bits_a = jnp.where(lo_a | hi_a, 0, jnp.int32(1) << iota)
            bits_b = jnp.where(lo_b | hi_b, 0, jnp.int32(1) << (iota + 16))
            aflags[pl.ds(k * 16, 16)] = bits_a | bits_b
            return carry2

        lax.fori_loop(0, _K, board_body, 0)

    def finish_stage(ch, rows, gsem, osem, aflags):
        b0 = base + ch * _K
        for k in range(_K):
            pltpu.make_async_copy(
                table_hbm.at[pl.ds(0, 16)],
                rows.at[pl.ds(k * _NUM_PIECES, 16)], gsem).wait()
            pltpu.make_async_copy(
                table_hbm.at[pl.ds(0, 16)],
                rows.at[pl.ds(k * _NUM_PIECES + 16, 16)], gsem).wait()

        def absent_fix(k, carry3):
            av = aflags[pl.ds(k * 16, 16)]
            bits = jnp.sum(av)
            for t in range(_NUM_PIECES):
                flag = (bits >> t) & 1

                @pl.when(flag != 0)
                def _zero_row():
                    r = k * _NUM_PIECES + t
                    for c0 in range(0, _C, 16):
                        rows[r, pl.ds(c0, 16)] = zf16
            return carry3

        lax.fori_loop(0, _K, absent_fix, 0)
        pltpu.async_copy(rows,
                         out_hbm.at[pl.ds(b0 * _NUM_PIECES,
                                          _K * _NUM_PIECES)], osem)

    def wait_out(rows, osem):
        pltpu.make_async_copy(
            table_hbm.at[pl.ds(0, _K * _NUM_PIECES)], rows, osem).wait()

    def pair_body(i2, carry):
        @pl.when(i2 > 0)
        def _w0():
            wait_out(rows0, osem0)

        issue_stage(2 * i2, rows0, gsem0, afl0)

        @pl.when(i2 > 0)
        def _f1():
            finish_stage(2 * i2 - 1, rows1, gsem1, osem1, afl1)
            wait_out(rows1, osem1)

        issue_stage(2 * i2 + 1, rows1, gsem1, afl1)
        finish_stage(2 * i2, rows0, gsem0, osem0, afl0)
        return carry

    lax.fori_loop(0, nch // 2, pair_body, 0)
    finish_stage(nch - 1, rows1, gsem1, osem1, afl1)
    wait_out(rows0, osem0)
    wait_out(rows1, osem1)


def kernel(full_board_vector, piece_ids):
    B, C, H, W = full_board_vector.shape
    table = full_board_vector.transpose(0, 2, 3, 1).reshape(B * H * W, C)
    ids_flat = piece_ids.reshape(B * H * W)

    mesh = plsc.VectorSubcoreMesh(core_axis_name="c", subcore_axis_name="s")
    run = functools.partial(
        pl.kernel,
        out_type=jax.ShapeDtypeStruct((B * _NUM_PIECES, C), jnp.float32),
        mesh=mesh,
        compiler_params=pltpu.CompilerParams(needs_layout_passes=False),
        scratch_types=[
            pltpu.VMEM((B * H * W // _NW,), jnp.int32),
            pltpu.VMEM((_K * _NUM_PIECES, _C), jnp.float32),
            pltpu.VMEM((_K * _NUM_PIECES, _C), jnp.float32),
            pltpu.VMEM((48,), jnp.int32),
            pltpu.VMEM((48,), jnp.int32),
            pltpu.VMEM((_K * 16,), jnp.int32),
            pltpu.VMEM((_K * 16,), jnp.int32),
            pltpu.SemaphoreType.DMA,
            pltpu.SemaphoreType.DMA,
            pltpu.SemaphoreType.DMA,
            pltpu.SemaphoreType.DMA,
        ],
    )(_sc_body)
    return run(table, ids_flat).reshape(B, _NUM_PIECES, C)
